# baseline (speedup 1.0000x reference)
import jax
import jax.numpy as jnp
from jax import lax
from jax.experimental import pallas as pl
from jax.experimental.pallas import tpu as pltpu
from jax.experimental.pallas import tpu_sc as plsc

V = 100000
C = 64
K = 8
B = 1024
L = 200

NC, NS, LANES = 2, 16, 16
NW = NC * NS
NBS = B // 128
NLG = NW // NBS
LPW = L // NLG
NCHUNK = LPW
TPC = 128
NG = TPC // LANES

_SORT8 = [(0, 1), (2, 3), (4, 5), (6, 7), (0, 2), (1, 3), (4, 6), (5, 7),
          (1, 2), (5, 6), (0, 4), (3, 7), (1, 5), (2, 6), (1, 4), (3, 6),
          (2, 4), (3, 5), (3, 4)]
_BITONIC8 = [(0, 4), (1, 5), (2, 6), (3, 7), (0, 2), (1, 3), (4, 6), (5, 7),
             (0, 1), (2, 3), (4, 5), (6, 7)]


def _ce(v, a, b):
    hi = jnp.maximum(v[a], v[b])
    lo = jnp.minimum(v[a], v[b])
    v[a], v[b] = hi, lo


def _top8_keys(keys):
    groups = []
    for g in range(8):
        gv = [keys[8 * g + i] for i in range(8)]
        for a, b in _SORT8:
            _ce(gv, a, b)
        groups.append(gv)
    while len(groups) > 1:
        merged = []
        for i in range(0, len(groups), 2):
            av, bv = groups[i], groups[i + 1]
            cv = [jnp.maximum(av[j], bv[7 - j]) for j in range(8)]
            for a, b in _BITONIC8:
                _ce(cv, a, b)
            merged.append(cv)
        groups = merged
    return groups[0]


def _sc_body(ids_hbm, wsw_hbm, wmu_hbm, wvar_hbm, tvar_hbm,
             score_out, mu_out, var_out, tvar_out,
             ids_blk, ids_t, sw0, mu0, var0, sw1, mu1, var1,
             score_st, mu_st, var_st, tv_v, tv_st, sem):
    w = lax.axis_index("s") * NC + lax.axis_index("c")
    sb = w % NBS
    boff = sb * 128
    lbase = (w // NBS) * LPW
    lane = jnp.arange(LANES, dtype=jnp.int32)
    bufs = ((sw0, mu0, var0), (sw1, mu1, var1))
    tables = (wsw_hbm, wmu_hbm, wvar_hbm)

    pltpu.sync_copy(ids_hbm.at[pl.ds(boff, 128), :], ids_blk)

    @pl.loop(0, LPW)
    def _transpose_ids(l):
        col = jnp.full((LANES,), 0, jnp.int32) + lbase + l
        for h in range(8):
            ids_t[l, pl.ds(h * 16, 16)] = plsc.load_gather(
                ids_blk, [h * 16 + lane, col])

    pltpu.sync_copy(tvar_hbm.at[pl.ds(w * 128, 128), :], tv_v)
    for i in range(8):
        r16 = i * 16 + lane
        a = plsc.load_gather(tv_v, [r16, jnp.full((LANES,), 0, jnp.int32)])
        bb = plsc.load_gather(tv_v, [r16, jnp.full((LANES,), 1, jnp.int32)])
        cc = plsc.load_gather(tv_v, [r16, jnp.full((LANES,), 2, jnp.int32)])
        o01 = bb * cc
        sl = pl.ds(i * 16, 16)
        tv_st[0, sl] = a * a + bb * bb
        tv_st[1, sl] = o01
        tv_st[2, sl] = o01
        tv_st[3, sl] = cc * cc
    pltpu.sync_copy(tv_st, tvar_out.at[:, pl.ds(w * 128, 128)])

    def fire(ci, slot):
        idx = ids_t.at[ci]
        for t in range(3):
            pltpu.async_copy(tables[t].at[idx], bufs[slot][t], sem)

    def drain(slot):
        for t in range(3):
            pltpu.make_async_copy(tables[t].at[pl.ds(0, TPC)],
                                  bufs[slot][t], sem).wait()

    def compute(ci, slot):
        swb, mub, varb = bufs[slot]

        @pl.loop(0, NG)
        def _group(g):
            off = g * 16
            tidx = off + lane
            keys = []
            for i in range(C):
                col = (lane + i) & 63
                v = plsc.load_gather(swb, [tidx, col])
                u = plsc.bitcast(v, jnp.int32)
                keys.append(plsc.bitcast((u | 63) - col, jnp.float32))
            kv = _top8_keys(keys)
            for k in range(K):
                ki = plsc.bitcast(kv[k], jnp.int32)
                idx = 63 - (ki & 63)
                sl = pl.ds(off, 16)
                score_st[k, sl] = plsc.load_gather(swb, [tidx, idx])
                mu_st[k, sl] = plsc.load_gather(mub, [tidx, idx])
                vv = plsc.load_gather(varb, [tidx, idx])
                var_st[k, sl] = vv * vv

        l0 = lbase + ci
        pltpu.sync_copy(score_st, score_out.at[l0, sb])
        pltpu.sync_copy(mu_st, mu_out.at[l0, sb])
        pltpu.sync_copy(var_st, var_out.at[l0, sb])

    fire(0, 0)

    @pl.loop(0, NCHUNK // 2 - 1)
    def _iter(it):
        ci0 = 2 * it
        fire(ci0 + 1, 1)
        drain(0)
        compute(ci0, 0)
        fire(ci0 + 2, 0)
        drain(1)
        compute(ci0 + 1, 1)

    fire(NCHUNK - 1, 1)
    drain(0)
    compute(NCHUNK - 2, 0)
    drain(1)
    compute(NCHUNK - 1, 1)


@jax.jit
def _sc_call(ids, wsw, wmu, wvar, tvar_r):
    mesh = plsc.VectorSubcoreMesh(core_axis_name="c", subcore_axis_name="s",
                                  num_cores=NC, num_subcores=NS)
    kern = pl.kernel(
        _sc_body,
        out_type=(
            jax.ShapeDtypeStruct((L, NBS, K, 128), jnp.float32),
            jax.ShapeDtypeStruct((L, NBS, K, 128), jnp.float32),
            jax.ShapeDtypeStruct((L, NBS, K, 128), jnp.float32),
            jax.ShapeDtypeStruct((4, C * C), jnp.float32),
        ),
        mesh=mesh,
        compiler_params=pltpu.CompilerParams(needs_layout_passes=False,
                                             use_tc_tiling_on_sc=False),
        scratch_types=[
            pltpu.VMEM((128, L), jnp.int32),
            pltpu.VMEM((LPW, 128), jnp.int32),
            pltpu.VMEM((TPC, C), jnp.float32),
            pltpu.VMEM((TPC, C), jnp.float32),
            pltpu.VMEM((TPC, C), jnp.float32),
            pltpu.VMEM((TPC, C), jnp.float32),
            pltpu.VMEM((TPC, C), jnp.float32),
            pltpu.VMEM((TPC, C), jnp.float32),
            pltpu.VMEM((K, 128), jnp.float32),
            pltpu.VMEM((K, 128), jnp.float32),
            pltpu.VMEM((K, 128), jnp.float32),
            pltpu.VMEM((128, 3), jnp.float32),
            pltpu.VMEM((4, 128), jnp.float32),
            pltpu.SemaphoreType.DMA,
        ],
    )
    return kern(ids, wsw, wmu, wvar, tvar_r)


def kernel(input, W_sweight, W_smu, W_svar, trans_mat_weight, trans_mat_mu,
           trans_mat_var):
    ids = input.astype(jnp.int32)
    tvar_r = trans_mat_var.reshape(C * C, 3)
    score4, mu4, var4, tvar_pl = _sc_call(ids, W_sweight, W_smu, W_svar,
                                          tvar_r)
    score = score4.transpose(0, 2, 1, 3).reshape(L, K, B)
    mu = mu4.transpose(0, 2, 1, 3).reshape(L, K, B)
    var = var4.transpose(0, 2, 1, 3).reshape(L, K, B)
    t_weight = trans_mat_weight.reshape(1, C, C, 1)
    t_mu = trans_mat_mu.reshape(1, C, C, 2)
    t_var = tvar_pl.T.reshape(1, C, C, 2, 2)
    return (score, mu, var, t_weight, t_mu, t_var)
